# Initial kernel scaffold; baseline (speedup 1.0000x reference)
import jax
import jax.numpy as jnp
from jax.experimental import pallas as pl


def kernel(f_obj, f_rel, f_frame, f_action, order_idx, W_obj, b_obj, W_rel, b_rel, W_frame, b_frame, W_action, b_action, order_table, pos_table, ln_g, ln_b):
    raise NotImplementedError("write your pallas kernel here")


# speedup vs baseline: 1.0045x; 1.0045x over previous
import functools

import jax
import jax.numpy as jnp
from jax import lax
from jax.experimental import pallas as pl
from jax.experimental.pallas import tpu as pltpu
from jax.experimental.pallas import tpu_sc as plsc

B = 8
N_OBJ = 1024
N_REL = 512
N_FRAME = 256
N_ACTION = 256
T = N_OBJ + N_REL + N_FRAME + N_ACTION
D = 512
H = 256
EPS = 1e-12
BLK = 256
CH = 128


def _sc_gather(table, idx_flat):
    info = plsc.get_sparse_core_info()
    nc, ns = info.num_cores, info.num_subcores
    nw = nc * ns
    n = idx_flat.shape[0]
    rows_per_w = n // nw
    n_chunks = rows_per_w // CH

    @functools.partial(
        pl.kernel,
        out_type=jax.ShapeDtypeStruct((n, H), jnp.float32),
        mesh=plsc.VectorSubcoreMesh(core_axis_name="c", subcore_axis_name="s"),
        scratch_types=[
            pltpu.VMEM((CH,), jnp.int32),
            pltpu.VMEM((CH, H), jnp.float32),
            pltpu.SemaphoreType.DMA,
        ],
    )
    def gather_kernel(table_hbm, idx_hbm, out_hbm, idx_v, rows_v, sem):
        wid = lax.axis_index("s") * nc + lax.axis_index("c")
        base = wid * rows_per_w
        for c in range(n_chunks):
            off = base + c * CH
            pltpu.sync_copy(idx_hbm.at[pl.ds(off, CH)], idx_v)
            pltpu.async_copy(table_hbm.at[idx_v], rows_v, sem).wait()
            pltpu.sync_copy(rows_v, out_hbm.at[pl.ds(off, CH)])

    return gather_kernel(table, idx_flat)


def _tc_body(fo, fr, ff, fa, w_ref, b_ref, oe_ref, g_ref, lb_ref, out_ref):
    j = pl.program_id(1)
    t = ((j >= 4).astype(jnp.int32) + (j >= 6).astype(jnp.int32)
         + (j >= 7).astype(jnp.int32))
    x = lax.switch(t, [lambda: fo[0], lambda: fr[0], lambda: ff[0],
                       lambda: fa[0]])
    w = lax.switch(t, [lambda: w_ref[0], lambda: w_ref[1], lambda: w_ref[2],
                       lambda: w_ref[3]])
    bias = lax.switch(t, [lambda: b_ref[0], lambda: b_ref[1],
                          lambda: b_ref[2], lambda: b_ref[3]])
    y = jnp.dot(x, w, preferred_element_type=jnp.float32) + bias + oe_ref[0]
    mu = jnp.mean(y, axis=-1, keepdims=True)
    var = jnp.mean((y - mu) ** 2, axis=-1, keepdims=True)
    out_ref[0] = (y - mu) * lax.rsqrt(var + EPS) * g_ref[...] + lb_ref[...]


def _tc_fused(fo, fr, ff, fa, w_stack, b_stack, order_emb, ln_g, ln_b):
    grid = (B, T // BLK)
    return pl.pallas_call(
        _tc_body,
        grid=grid,
        in_specs=[
            pl.BlockSpec((1, BLK, D), lambda b, j: (b, jnp.minimum(j, 3), 0)),
            pl.BlockSpec((1, BLK, D),
                         lambda b, j: (b, jnp.clip(j - 4, 0, 1), 0)),
            pl.BlockSpec((1, BLK, D), lambda b, j: (b, 0, 0)),
            pl.BlockSpec((1, BLK, D), lambda b, j: (b, 0, 0)),
            pl.BlockSpec((4, D, H), lambda b, j: (0, 0, 0)),
            pl.BlockSpec((4, 1, H), lambda b, j: (0, 0, 0)),
            pl.BlockSpec((1, BLK, H), lambda b, j: (b, j, 0)),
            pl.BlockSpec((1, H), lambda b, j: (0, 0)),
            pl.BlockSpec((1, H), lambda b, j: (0, 0)),
        ],
        out_specs=pl.BlockSpec((1, BLK, H), lambda b, j: (b, j, 0)),
        out_shape=jax.ShapeDtypeStruct((B, T, H), jnp.float32),
    )(fo, fr, ff, fa, w_stack, b_stack, order_emb, ln_g, ln_b)


def kernel(f_obj, f_rel, f_frame, f_action, order_idx, W_obj, b_obj, W_rel,
           b_rel, W_frame, b_frame, W_action, b_action, order_table,
           pos_table, ln_g, ln_b):
    idx_flat = order_idx.reshape(-1).astype(jnp.int32)
    order_emb = _sc_gather(order_table, idx_flat).reshape(B, T, H)
    w_stack = jnp.stack([W_obj, W_rel, W_frame, W_action])
    b_stack = jnp.stack([b_obj, b_rel, b_frame, b_action]).reshape(4, 1, H)
    out = _tc_fused(
        f_obj.reshape(B, N_OBJ, D),
        f_rel.reshape(B, N_REL, D),
        f_frame.reshape(B, N_FRAME, D),
        f_action.reshape(B, N_ACTION, D),
        w_stack, b_stack, order_emb,
        ln_g.reshape(1, H), ln_b.reshape(1, H),
    )
    non_pad_mask = jnp.ones((B, T), dtype=bool)
    return (out, non_pad_mask)


# speedup vs baseline: 1.1848x; 1.1794x over previous
import functools

import jax
import jax.numpy as jnp
from jax import lax
from jax.experimental import pallas as pl
from jax.experimental.pallas import tpu as pltpu
from jax.experimental.pallas import tpu_sc as plsc

B = 8
N_OBJ = 1024
N_REL = 512
N_FRAME = 256
N_ACTION = 256
T = N_OBJ + N_REL + N_FRAME + N_ACTION
D = 512
H = 256
EPS = 1e-12
BLK = 256
CH = 128


def _sc_gather(table, idx_flat):
    info = plsc.get_sparse_core_info()
    nc, ns = info.num_cores, info.num_subcores
    nw = nc * ns
    n = idx_flat.shape[0]
    rows_per_w = n // nw
    n_chunks = rows_per_w // CH

    @functools.partial(
        pl.kernel,
        out_type=jax.ShapeDtypeStruct((n, H), jnp.float32),
        mesh=plsc.VectorSubcoreMesh(core_axis_name="c", subcore_axis_name="s"),
        scratch_types=[
            pltpu.VMEM((CH,), jnp.int32),
            pltpu.VMEM((CH, H), jnp.float32),
            pltpu.SemaphoreType.DMA,
        ],
    )
    def gather_kernel(table_hbm, idx_hbm, out_hbm, idx_v, rows_v, sem):
        wid = lax.axis_index("s") * nc + lax.axis_index("c")
        base = wid * rows_per_w
        for c in range(n_chunks):
            off = base + c * CH
            pltpu.sync_copy(idx_hbm.at[pl.ds(off, CH)], idx_v)
            pltpu.async_copy(table_hbm.at[idx_v], rows_v, sem).wait()
            pltpu.sync_copy(rows_v, out_hbm.at[pl.ds(off, CH)])

    return gather_kernel(table, idx_flat)


def _tc_body(fo, fr, ff, fa, w_ref, b_ref, oe_ref, g_ref, lb_ref, out_ref):
    j = pl.program_id(1)
    t = ((j >= 4).astype(jnp.int32) + (j >= 6).astype(jnp.int32)
         + (j >= 7).astype(jnp.int32))

    def emit(x, k):
        y = (jnp.dot(x, w_ref[k], preferred_element_type=jnp.float32)
             + b_ref[k] + oe_ref[0])
        mu = jnp.mean(y, axis=-1, keepdims=True)
        var = jnp.mean((y - mu) ** 2, axis=-1, keepdims=True)
        out_ref[0] = (y - mu) * lax.rsqrt(var + EPS) * g_ref[...] + lb_ref[...]

    @pl.when(t == 0)
    def _():
        emit(fo[0], 0)

    @pl.when(t == 1)
    def _():
        emit(fr[0], 1)

    @pl.when(t == 2)
    def _():
        emit(ff[0], 2)

    @pl.when(t == 3)
    def _():
        emit(fa[0], 3)


def _tc_fused(fo, fr, ff, fa, w_stack, b_stack, order_emb, ln_g, ln_b):
    grid = (B, T // BLK)
    return pl.pallas_call(
        _tc_body,
        grid=grid,
        in_specs=[
            pl.BlockSpec((1, BLK, D), lambda b, j: (b, jnp.minimum(j, 3), 0)),
            pl.BlockSpec((1, BLK, D),
                         lambda b, j: (b, jnp.clip(j - 4, 0, 1), 0)),
            pl.BlockSpec((1, BLK, D), lambda b, j: (b, 0, 0)),
            pl.BlockSpec((1, BLK, D), lambda b, j: (b, 0, 0)),
            pl.BlockSpec((4, D, H), lambda b, j: (0, 0, 0)),
            pl.BlockSpec((4, 1, H), lambda b, j: (0, 0, 0)),
            pl.BlockSpec((1, BLK, H), lambda b, j: (b, j, 0)),
            pl.BlockSpec((1, H), lambda b, j: (0, 0)),
            pl.BlockSpec((1, H), lambda b, j: (0, 0)),
        ],
        out_specs=pl.BlockSpec((1, BLK, H), lambda b, j: (b, j, 0)),
        out_shape=jax.ShapeDtypeStruct((B, T, H), jnp.float32),
    )(fo, fr, ff, fa, w_stack, b_stack, order_emb, ln_g, ln_b)


def kernel(f_obj, f_rel, f_frame, f_action, order_idx, W_obj, b_obj, W_rel,
           b_rel, W_frame, b_frame, W_action, b_action, order_table,
           pos_table, ln_g, ln_b):
    idx_flat = order_idx.reshape(-1).astype(jnp.int32)
    order_emb = _sc_gather(order_table, idx_flat).reshape(B, T, H)
    w_stack = jnp.stack([W_obj, W_rel, W_frame, W_action])
    b_stack = jnp.stack([b_obj, b_rel, b_frame, b_action]).reshape(4, 1, H)
    out = _tc_fused(
        f_obj.reshape(B, N_OBJ, D),
        f_rel.reshape(B, N_REL, D),
        f_frame.reshape(B, N_FRAME, D),
        f_action.reshape(B, N_ACTION, D),
        w_stack, b_stack, order_emb,
        ln_g.reshape(1, H), ln_b.reshape(1, H),
    )
    non_pad_mask = jnp.ones((B, T), dtype=bool)
    return (out, non_pad_mask)


# speedup vs baseline: 1.2850x; 1.0846x over previous
import functools

import jax
import jax.numpy as jnp
from jax import lax
from jax.experimental import pallas as pl
from jax.experimental.pallas import tpu as pltpu
from jax.experimental.pallas import tpu_sc as plsc

B = 8
N_OBJ = 1024
N_REL = 512
N_FRAME = 256
N_ACTION = 256
T = N_OBJ + N_REL + N_FRAME + N_ACTION
D = 512
H = 256
EPS = 1e-12
BLK = 256
CH = 128


def _sc_gather(table_words, idx_flat):
    hw = table_words.shape[1]
    info = plsc.get_sparse_core_info()
    nc, ns = info.num_cores, info.num_subcores
    nw = nc * ns
    n = idx_flat.shape[0]
    rows_per_w = n // nw
    n_chunks = rows_per_w // CH

    @functools.partial(
        pl.kernel,
        out_type=jax.ShapeDtypeStruct((n, hw), jnp.int32),
        mesh=plsc.VectorSubcoreMesh(core_axis_name="c", subcore_axis_name="s"),
        scratch_types=[
            pltpu.VMEM((rows_per_w,), jnp.int32),
            [pltpu.VMEM((CH, hw), jnp.int32) for _ in range(n_chunks)],
            [pltpu.SemaphoreType.DMA for _ in range(n_chunks)],
            [pltpu.SemaphoreType.DMA for _ in range(n_chunks)],
        ],
    )
    def gather_kernel(table_hbm, idx_hbm, out_hbm, idx_v, rows_v, gsems, wsems):
        wid = lax.axis_index("s") * nc + lax.axis_index("c")
        base = wid * rows_per_w
        pltpu.sync_copy(idx_hbm.at[pl.ds(base, rows_per_w)], idx_v)
        gathers = [
            pltpu.async_copy(
                table_hbm.at[idx_v.at[pl.ds(c * CH, CH)]], rows_v[c], gsems[c])
            for c in range(n_chunks)
        ]
        writes = []
        for c in range(n_chunks):
            gathers[c].wait()
            writes.append(pltpu.async_copy(
                rows_v[c], out_hbm.at[pl.ds(base + c * CH, CH)], wsems[c]))
        for w in writes:
            w.wait()

    return gather_kernel(table_words, idx_flat)


def _tc_body(fo, fr, ff, fa, w_ref, b_ref, oe_ref, g_ref, lb_ref, out_ref):
    j = pl.program_id(1)
    t = ((j >= 4).astype(jnp.int32) + (j >= 6).astype(jnp.int32)
         + (j >= 7).astype(jnp.int32))

    def emit(x, k):
        w = oe_ref[0]
        lo = lax.bitcast_convert_type(w << 16, jnp.float32)
        hi = lax.bitcast_convert_type(w & jnp.int32(-65536), jnp.float32)
        oe = jnp.concatenate([lo, hi], axis=-1)
        y = (jnp.dot(x, w_ref[k], preferred_element_type=jnp.float32)
             + b_ref[k] + oe)
        mu = jnp.mean(y, axis=-1, keepdims=True)
        var = jnp.mean((y - mu) ** 2, axis=-1, keepdims=True)
        out_ref[0] = (y - mu) * lax.rsqrt(var + EPS) * g_ref[...] + lb_ref[...]

    @pl.when(t == 0)
    def _():
        emit(fo[0], 0)

    @pl.when(t == 1)
    def _():
        emit(fr[0], 1)

    @pl.when(t == 2)
    def _():
        emit(ff[0], 2)

    @pl.when(t == 3)
    def _():
        emit(fa[0], 3)


def _tc_fused(fo, fr, ff, fa, w_stack, b_stack, order_emb, ln_g, ln_b):
    grid = (B, T // BLK)
    return pl.pallas_call(
        _tc_body,
        grid=grid,
        in_specs=[
            pl.BlockSpec((1, BLK, D), lambda b, j: (b, jnp.minimum(j, 3), 0)),
            pl.BlockSpec((1, BLK, D),
                         lambda b, j: (b, jnp.clip(j - 4, 0, 1), 0)),
            pl.BlockSpec((1, BLK, D), lambda b, j: (b, 0, 0)),
            pl.BlockSpec((1, BLK, D), lambda b, j: (b, 0, 0)),
            pl.BlockSpec((4, D, H), lambda b, j: (0, 0, 0)),
            pl.BlockSpec((4, 1, H), lambda b, j: (0, 0, 0)),
            pl.BlockSpec((1, BLK, H // 2), lambda b, j: (b, j, 0)),
            pl.BlockSpec((1, H), lambda b, j: (0, 0)),
            pl.BlockSpec((1, H), lambda b, j: (0, 0)),
        ],
        out_specs=pl.BlockSpec((1, BLK, H), lambda b, j: (b, j, 0)),
        out_shape=jax.ShapeDtypeStruct((B, T, H), jnp.float32),
    )(fo, fr, ff, fa, w_stack, b_stack, order_emb, ln_g, ln_b)


def kernel(f_obj, f_rel, f_frame, f_action, order_idx, W_obj, b_obj, W_rel,
           b_rel, W_frame, b_frame, W_action, b_action, order_table,
           pos_table, ln_g, ln_b):
    idx_flat = order_idx.reshape(-1).astype(jnp.int32)
    tb = order_table.astype(jnp.bfloat16)
    tb_words = jax.lax.bitcast_convert_type(
        jnp.stack([tb[:, :H // 2], tb[:, H // 2:]], axis=-1), jnp.int32)
    order_emb = _sc_gather(tb_words, idx_flat).reshape(B, T, H // 2)
    w_stack = jnp.stack([W_obj, W_rel, W_frame, W_action])
    b_stack = jnp.stack([b_obj, b_rel, b_frame, b_action]).reshape(4, 1, H)
    out = _tc_fused(
        f_obj.reshape(B, N_OBJ, D),
        f_rel.reshape(B, N_REL, D),
        f_frame.reshape(B, N_FRAME, D),
        f_action.reshape(B, N_ACTION, D),
        w_stack, b_stack, order_emb,
        ln_g.reshape(1, H), ln_b.reshape(1, H),
    )
    non_pad_mask = jnp.ones((B, T), dtype=bool)
    return (out, non_pad_mask)


# speedup vs baseline: 1.2919x; 1.0054x over previous
import functools

import jax
import jax.numpy as jnp
from jax import lax
from jax.experimental import pallas as pl
from jax.experimental.pallas import tpu as pltpu
from jax.experimental.pallas import tpu_sc as plsc

B = 8
N_OBJ = 1024
N_REL = 512
N_FRAME = 256
N_ACTION = 256
T = N_OBJ + N_REL + N_FRAME + N_ACTION
D = 512
H = 256
EPS = 1e-12
BLK = 256
CH = 128


def _sc_gather(table_words, idx_flat):
    hw = table_words.shape[1]
    info = plsc.get_sparse_core_info()
    nc, ns = info.num_cores, info.num_subcores
    nw = nc * ns
    n = idx_flat.shape[0]
    rows_per_w = n // nw
    n_chunks = rows_per_w // CH

    @functools.partial(
        pl.kernel,
        out_type=jax.ShapeDtypeStruct((n, hw), jnp.int32),
        mesh=plsc.VectorSubcoreMesh(core_axis_name="c", subcore_axis_name="s"),
        scratch_types=[
            pltpu.VMEM((rows_per_w,), jnp.int32),
            [pltpu.VMEM((CH, hw), jnp.int32) for _ in range(n_chunks)],
            [pltpu.SemaphoreType.DMA for _ in range(n_chunks)],
            [pltpu.SemaphoreType.DMA for _ in range(n_chunks)],
        ],
    )
    def gather_kernel(table_hbm, idx_hbm, out_hbm, idx_v, rows_v, gsems, wsems):
        wid = lax.axis_index("s") * nc + lax.axis_index("c")
        base = wid * rows_per_w
        pltpu.sync_copy(idx_hbm.at[pl.ds(base, rows_per_w)], idx_v)
        gathers = [
            pltpu.async_copy(
                table_hbm.at[idx_v.at[pl.ds(c * CH, CH)]], rows_v[c], gsems[c])
            for c in range(n_chunks)
        ]
        writes = []
        for c in range(n_chunks):
            gathers[c].wait()
            writes.append(pltpu.async_copy(
                rows_v[c], out_hbm.at[pl.ds(base + c * CH, CH)], wsems[c]))
        for w in writes:
            w.wait()

    return gather_kernel(table_words, idx_flat)


def _tc_body(fo, fr, ff, fa, w_ref, b_ref, oe_ref, g_ref, lb_ref, out_ref):
    j = pl.program_id(1)
    t = ((j >= 4).astype(jnp.int32) + (j >= 6).astype(jnp.int32)
         + (j >= 7).astype(jnp.int32))

    def emit(x, k):
        w = oe_ref[0]
        lo = lax.bitcast_convert_type(w << 16, jnp.float32)
        hi = lax.bitcast_convert_type(w & jnp.int32(-65536), jnp.float32)
        oe = jnp.concatenate([lo, hi], axis=-1)
        y = (jnp.dot(x, w_ref[k], preferred_element_type=jnp.float32)
             + b_ref[k] + oe)
        mu = jnp.mean(y, axis=-1, keepdims=True)
        var = jnp.mean((y - mu) ** 2, axis=-1, keepdims=True)
        out_ref[0] = (y - mu) * lax.rsqrt(var + EPS) * g_ref[...] + lb_ref[...]

    @pl.when(t == 0)
    def _():
        emit(fo[0], 0)

    @pl.when(t == 1)
    def _():
        emit(fr[0], 1)

    @pl.when(t == 2)
    def _():
        emit(ff[0], 2)

    @pl.when(t == 3)
    def _():
        emit(fa[0], 3)


def _tc_half(fo, fr, ff, fa, w_stack, b_stack, oe_half, ln_g, ln_b, b_base,
             out_init):
    nb = oe_half.shape[0]
    in_specs = [
        pl.BlockSpec((1, BLK, D),
                     lambda b, j: (b + b_base, jnp.minimum(j, 3), 0)),
        pl.BlockSpec((1, BLK, D),
                     lambda b, j: (b + b_base, jnp.clip(j - 4, 0, 1), 0)),
        pl.BlockSpec((1, BLK, D), lambda b, j: (b + b_base, 0, 0)),
        pl.BlockSpec((1, BLK, D), lambda b, j: (b + b_base, 0, 0)),
        pl.BlockSpec((4, D, H), lambda b, j: (0, 0, 0)),
        pl.BlockSpec((4, 1, H), lambda b, j: (0, 0, 0)),
        pl.BlockSpec((1, BLK, H // 2), lambda b, j: (b, j, 0)),
        pl.BlockSpec((1, H), lambda b, j: (0, 0)),
        pl.BlockSpec((1, H), lambda b, j: (0, 0)),
    ]
    args = [fo, fr, ff, fa, w_stack, b_stack, oe_half, ln_g, ln_b]
    body = _tc_body
    aliases = {}
    if out_init is not None:
        in_specs.append(pl.BlockSpec(memory_space=pl.ANY))
        args.append(out_init)
        aliases = {9: 0}
        body = lambda *refs: _tc_body(*refs[:9], refs[10])
    return pl.pallas_call(
        body,
        grid=(nb, T // BLK),
        in_specs=in_specs,
        out_specs=pl.BlockSpec((1, BLK, H),
                               lambda b, j: (b + b_base, j, 0)),
        out_shape=jax.ShapeDtypeStruct((B, T, H), jnp.float32),
        input_output_aliases=aliases,
    )(*args)


def kernel(f_obj, f_rel, f_frame, f_action, order_idx, W_obj, b_obj, W_rel,
           b_rel, W_frame, b_frame, W_action, b_action, order_table,
           pos_table, ln_g, ln_b):
    idx_flat = order_idx.reshape(-1).astype(jnp.int32)
    tb = order_table.astype(jnp.bfloat16)
    tb_words = jax.lax.bitcast_convert_type(
        jnp.stack([tb[:, :H // 2], tb[:, H // 2:]], axis=-1), jnp.int32)
    half = B // 2
    oe_a = _sc_gather(tb_words, idx_flat[:half * T]).reshape(half, T, H // 2)
    oe_b = _sc_gather(tb_words, idx_flat[half * T:]).reshape(half, T, H // 2)
    w_stack = jnp.stack([W_obj, W_rel, W_frame, W_action])
    b_stack = jnp.stack([b_obj, b_rel, b_frame, b_action]).reshape(4, 1, H)
    fo = f_obj.reshape(B, N_OBJ, D)
    fr = f_rel.reshape(B, N_REL, D)
    ff = f_frame.reshape(B, N_FRAME, D)
    fa = f_action.reshape(B, N_ACTION, D)
    g2 = ln_g.reshape(1, H)
    lb2 = ln_b.reshape(1, H)
    out = _tc_half(fo, fr, ff, fa, w_stack, b_stack, oe_a, g2, lb2, 0, None)
    out = _tc_half(fo, fr, ff, fa, w_stack, b_stack, oe_b, g2, lb2, half, out)
    non_pad_mask = jnp.ones((B, T), dtype=bool)
    return (out, non_pad_mask)
